# Initial kernel scaffold; baseline (speedup 1.0000x reference)
import jax
import jax.numpy as jnp
from jax.experimental import pallas as pl


def kernel(edge_index, features, preference, W):
    raise NotImplementedError("write your pallas kernel here")


# speedup vs baseline: 4.2428x; 4.2428x over previous
import functools

import jax
import jax.numpy as jnp
from jax import lax
from jax.experimental import pallas as pl
from jax.experimental.pallas import tpu as pltpu
from jax.experimental.pallas import tpu_sc as plsc

N_USER = 2000
N_ITEM = 8000
N_NODES = N_USER + N_ITEM
DIM = 128
N_EDGES = 320000

NC = 2
NS = 16
CHUNK = 128
CHUNKS_PER_WORKER = -(-N_EDGES // (NC * NS * CHUNK))
E_PAD = NC * NS * CHUNKS_PER_WORKER * CHUNK
ACC_ROWS = 10240
ROWS_PER_SUB = ACC_ROWS // NS
ZROWS = 64

_sc_mesh = plsc.VectorSubcoreMesh(core_axis_name="c", subcore_axis_name="s")


@functools.partial(
    pl.kernel,
    out_type=jax.ShapeDtypeStruct((NC, ACC_ROWS, DIM), jnp.float32),
    mesh=_sc_mesh,
    scratch_types=[
        pltpu.VMEM((CHUNK,), jnp.int32),
        pltpu.VMEM((CHUNK,), jnp.int32),
        pltpu.VMEM((CHUNK, DIM), jnp.float32),
        pltpu.VMEM((ZROWS, DIM), jnp.float32),
        pltpu.VMEM_SHARED((ACC_ROWS, DIM), jnp.float32),
    ],
)
def _sc_aggregate(xn_hbm, src_hbm, dst_hbm, out_hbm,
                  src_v, dst_v, rows_v, zero_v, acc_sh):
    cid = lax.axis_index("c")
    sid = lax.axis_index("s")

    @pl.loop(0, ZROWS)
    def _(r):
        @pl.loop(0, DIM, step=16)
        def _(q):
            zero_v[pl.ds(r, 1), pl.ds(q, 16)] = jnp.zeros((1, 16), jnp.float32)

    @pl.loop(0, ROWS_PER_SUB, step=ZROWS)
    def _(r):
        pltpu.sync_copy(zero_v, acc_sh.at[pl.ds(sid * ROWS_PER_SUB + r, ZROWS)])

    plsc.subcore_barrier()

    wid = cid * NS + sid
    base = wid * (CHUNKS_PER_WORKER * CHUNK)

    @pl.loop(0, CHUNKS_PER_WORKER)
    def _(i):
        off = base + i * CHUNK
        pltpu.sync_copy(src_hbm.at[pl.ds(off, CHUNK)], src_v)
        pltpu.sync_copy(dst_hbm.at[pl.ds(off, CHUNK)], dst_v)
        pltpu.sync_copy(xn_hbm.at[src_v], rows_v)
        pltpu.sync_copy(rows_v, acc_sh.at[dst_v], add=True)

    plsc.subcore_barrier()

    pltpu.sync_copy(acc_sh.at[pl.ds(sid * ROWS_PER_SUB, ROWS_PER_SUB)],
                    out_hbm.at[cid, pl.ds(sid * ROWS_PER_SUB, ROWS_PER_SUB)])


def _normalize_body(x_ref, o_ref):
    x = x_ref[...]
    s = jnp.sum(x * x, axis=1, keepdims=True)
    norm = jnp.sqrt(s)
    o_ref[...] = x / jnp.maximum(norm, 1e-12)


def _combine_body(p0_ref, p1_ref, xn_ref, w_ref, o_ref):
    s = p0_ref[0] + p1_ref[0]
    o_ref[...] = (
        jnp.dot(s, w_ref[...], preferred_element_type=jnp.float32)
        + xn_ref[...]
    )


_ROWB = 1000


def kernel(edge_index, features, preference, W):
    x = jnp.concatenate([preference, features], axis=0)
    src = edge_index[0]
    dst = edge_index[1]
    pad = E_PAD - N_EDGES
    src_p = jnp.concatenate([src, jnp.zeros((pad,), jnp.int32)])
    dst_p = jnp.concatenate([dst, jnp.full((pad,), N_NODES, jnp.int32)])

    xn = pl.pallas_call(
        _normalize_body,
        out_shape=jax.ShapeDtypeStruct((N_NODES, DIM), jnp.float32),
        grid=(N_NODES // _ROWB,),
        in_specs=[pl.BlockSpec((_ROWB, DIM), lambda i: (i, 0))],
        out_specs=pl.BlockSpec((_ROWB, DIM), lambda i: (i, 0)),
    )(x)

    parts = _sc_aggregate(xn, src_p, dst_p)

    x_hat = pl.pallas_call(
        _combine_body,
        out_shape=jax.ShapeDtypeStruct((N_NODES, DIM), jnp.float32),
        grid=(N_NODES // _ROWB,),
        in_specs=[
            pl.BlockSpec((1, _ROWB, DIM), lambda i: (0, i, 0)),
            pl.BlockSpec((1, _ROWB, DIM), lambda i: (1, i, 0)),
            pl.BlockSpec((_ROWB, DIM), lambda i: (i, 0)),
            pl.BlockSpec((DIM, DIM), lambda i: (0, 0)),
        ],
        out_specs=pl.BlockSpec((_ROWB, DIM), lambda i: (i, 0)),
    )(parts, parts, xn, W)

    return (x_hat, preference)
